# baseline (speedup 1.0000x reference)
import functools

import jax
import jax.numpy as jnp
from jax import lax
from jax.experimental import pallas as pl
from jax.experimental.pallas import tpu as pltpu
from jax.experimental.pallas import tpu_sc as plsc

N_NODES = 10000
N_EDGES = 320000
D = 128

NC = 2
NS = 16
NW = NC * NS
EDGES_PER_W = N_EDGES // NW
CHUNK = 400
N_CHUNKS = EDGES_PER_W // CHUNK


def _mm_bias_kernel(h_ref, w1t_ref, b_ref, o_ref):
    o_ref[...] = (
        jnp.dot(h_ref[...], w1t_ref[...], preferred_element_type=jnp.float32)
        + b_ref[...]
    )


def _node_transform(H, W1t, b2d):
    return pl.pallas_call(
        _mm_bias_kernel,
        out_shape=jax.ShapeDtypeStruct((N_NODES, D), jnp.float32),
    )(H, W1t, b2d)


def _sc_gather_body(table_hbm, idx_hbm, out_hbm, idx_v, rows_v, sem):
    wid = lax.axis_index("s") * NC + lax.axis_index("c")
    base = wid * EDGES_PER_W

    def body(i, carry):
        off = base + i * CHUNK
        pltpu.sync_copy(idx_hbm.at[pl.ds(off, CHUNK)], idx_v)
        pltpu.async_copy(table_hbm.at[idx_v], rows_v, sem).wait()
        pltpu.sync_copy(rows_v, out_hbm.at[pl.ds(off, CHUNK)])
        return carry

    lax.fori_loop(0, N_CHUNKS, body, 0)


def _sc_gather(table, heads):
    mesh = plsc.VectorSubcoreMesh(core_axis_name="c", subcore_axis_name="s")
    k = functools.partial(
        pl.kernel,
        mesh=mesh,
        out_type=jax.ShapeDtypeStruct((N_EDGES, D), jnp.float32),
        scratch_types=[
            pltpu.VMEM((CHUNK,), jnp.int32),
            pltpu.VMEM((CHUNK, D), jnp.float32),
            pltpu.SemaphoreType.DMA,
        ],
    )(_sc_gather_body)
    return k(table, heads)


def _add_mm_kernel(g_ref, e_ref, w2t_ref, o_ref):
    o_ref[...] = g_ref[...] + jnp.dot(
        e_ref[...], w2t_ref[...], preferred_element_type=jnp.float32
    )


def _edge_transform(G, E, W2t, blk):
    n_blocks = N_EDGES // blk
    return pl.pallas_call(
        _add_mm_kernel,
        grid=(n_blocks,),
        in_specs=[
            pl.BlockSpec((blk, D), lambda i: (i, 0)),
            pl.BlockSpec((blk, D), lambda i: (i, 0)),
            pl.BlockSpec((D, D), lambda i: (0, 0)),
        ],
        out_specs=pl.BlockSpec((blk, D), lambda i: (i, 0)),
        out_shape=jax.ShapeDtypeStruct((N_EDGES, D), jnp.float32),
    )(G, E, W2t)


@jax.jit
def kernel(H, E, heads, queries, W, b):
    W1t = W[:, :D].T
    W2t = W[:, D:].T
    b2d = b.reshape(1, D)
    T = _node_transform(H, W1t, b2d)
    G = _sc_gather(T, heads.astype(jnp.int32))
    return _edge_transform(G, E, W2t, blk=16000)


# speedup vs baseline: 1.0510x; 1.0510x over previous
import functools

import jax
import jax.numpy as jnp
from jax import lax
from jax.experimental import pallas as pl
from jax.experimental.pallas import tpu as pltpu
from jax.experimental.pallas import tpu_sc as plsc

N_NODES = 10000
N_EDGES = 320000
D = 128

NC = 2
NS = 16
NW = NC * NS
EDGES_PER_W = N_EDGES // NW
CHUNK = 200
N_CHUNKS = EDGES_PER_W // CHUNK
N_PAIRS = N_CHUNKS // 2


def _mm_bias_kernel(h_ref, w1t_ref, b_ref, o_ref):
    o_ref[...] = (
        jnp.dot(h_ref[...], w1t_ref[...], preferred_element_type=jnp.float32)
        + b_ref[...]
    )


def _node_transform(H, W1t, b2d):
    return pl.pallas_call(
        _mm_bias_kernel,
        out_shape=jax.ShapeDtypeStruct((N_NODES, D), jnp.float32),
    )(H, W1t, b2d)


def _sc_gather_body(
    table_hbm, idx_hbm, out_hbm, idx_all, rows0, rows1, gsem0, gsem1, osem0, osem1
):
    wid = lax.axis_index("s") * NC + lax.axis_index("c")
    base = wid * EDGES_PER_W

    def idx_slice(c):
        return idx_all.at[pl.ds(c * CHUNK, CHUNK)]

    def out_slice(c):
        return out_hbm.at[pl.ds(base + c * CHUNK, CHUNK)]

    pltpu.sync_copy(idx_hbm.at[pl.ds(base, EDGES_PER_W)], idx_all)

    def g_start(c, rows, sem):
        pltpu.async_copy(table_hbm.at[idx_slice(c)], rows, sem)

    def g_wait(c, rows, sem):
        pltpu.make_async_copy(table_hbm.at[idx_slice(c)], rows, sem).wait()

    def s_start(c, rows, sem):
        pltpu.async_copy(rows, out_slice(c), sem)

    def s_wait(c, rows, sem):
        pltpu.make_async_copy(rows, out_slice(c), sem).wait()

    g_start(0, rows0, gsem0)
    g_wait(0, rows0, gsem0)
    s_start(0, rows0, osem0)
    g_start(1, rows1, gsem1)
    g_wait(1, rows1, gsem1)
    s_start(1, rows1, osem1)
    s_wait(0, rows0, osem0)
    g_start(2, rows0, gsem0)

    def pair(j, carry):
        c0 = 2 * j
        c1 = c0 + 1
        g_wait(c0, rows0, gsem0)
        s_start(c0, rows0, osem0)
        s_wait(c1 - 2, rows1, osem1)
        g_start(c1, rows1, gsem1)
        g_wait(c1, rows1, gsem1)
        s_start(c1, rows1, osem1)
        s_wait(c0, rows0, osem0)
        g_start(c0 + 2, rows0, gsem0)
        return carry

    lax.fori_loop(1, N_PAIRS - 1, pair, 0)

    cz = N_CHUNKS - 2
    g_wait(cz, rows0, gsem0)
    s_start(cz, rows0, osem0)
    s_wait(cz - 1, rows1, osem1)
    g_start(cz + 1, rows1, gsem1)
    g_wait(cz + 1, rows1, gsem1)
    s_start(cz + 1, rows1, osem1)
    s_wait(cz, rows0, osem0)
    s_wait(cz + 1, rows1, osem1)


def _sc_gather(table, heads):
    mesh = plsc.VectorSubcoreMesh(core_axis_name="c", subcore_axis_name="s")
    k = functools.partial(
        pl.kernel,
        mesh=mesh,
        out_type=jax.ShapeDtypeStruct((N_EDGES, D), jnp.float32),
        scratch_types=[
            pltpu.VMEM((EDGES_PER_W,), jnp.int32),
            pltpu.VMEM((CHUNK, D), jnp.float32),
            pltpu.VMEM((CHUNK, D), jnp.float32),
            pltpu.SemaphoreType.DMA,
            pltpu.SemaphoreType.DMA,
            pltpu.SemaphoreType.DMA,
            pltpu.SemaphoreType.DMA,
        ],
    )(_sc_gather_body)
    return k(table, heads)


def _add_mm_kernel(g_ref, e_ref, w2t_ref, o_ref):
    o_ref[...] = g_ref[...] + jnp.dot(
        e_ref[...], w2t_ref[...], preferred_element_type=jnp.float32
    )


def _edge_transform(G, E, W2t, blk):
    n_blocks = N_EDGES // blk
    return pl.pallas_call(
        _add_mm_kernel,
        grid=(n_blocks,),
        in_specs=[
            pl.BlockSpec((blk, D), lambda i: (i, 0)),
            pl.BlockSpec((blk, D), lambda i: (i, 0)),
            pl.BlockSpec((D, D), lambda i: (0, 0)),
        ],
        out_specs=pl.BlockSpec((blk, D), lambda i: (i, 0)),
        out_shape=jax.ShapeDtypeStruct((N_EDGES, D), jnp.float32),
    )(G, E, W2t)


@jax.jit
def kernel(H, E, heads, queries, W, b):
    W1t = W[:, :D].T
    W2t = W[:, D:].T
    b2d = b.reshape(1, D)
    T = _node_transform(H, W1t, b2d)
    G = _sc_gather(T, heads.astype(jnp.int32))
    return _edge_transform(G, E, W2t, blk=8000)


# speedup vs baseline: 1.0899x; 1.0369x over previous
import functools

import jax
import jax.numpy as jnp
from jax import lax
from jax.experimental import pallas as pl
from jax.experimental.pallas import tpu as pltpu
from jax.experimental.pallas import tpu_sc as plsc

N_NODES = 10000
N_EDGES = 320000
D = 128

NC = 2
NS = 16
NW = NC * NS
EDGES_PER_W = N_EDGES // NW
CHUNK = 200
N_CHUNKS = EDGES_PER_W // CHUNK
N_TRIPLES = (N_CHUNKS - 2) // 3


def _mm_bias_kernel(h_ref, w_ref, b_ref, o_ref):
    w1 = w_ref[:, :D]
    o_ref[...] = (
        lax.dot_general(
            h_ref[...], w1, (((1,), (1,)), ((), ())),
            preferred_element_type=jnp.float32,
        )
        + b_ref[...]
    )


def _node_transform(H, W, b2d):
    return pl.pallas_call(
        _mm_bias_kernel,
        out_shape=jax.ShapeDtypeStruct((N_NODES, D), jnp.float32),
    )(H, W, b2d)


def _sc_gather_body(
    table_hbm, idx_hbm, out_hbm, idx_all, rows, gsems, osems
):
    wid = lax.axis_index("s") * NC + lax.axis_index("c")
    base = wid * EDGES_PER_W

    def idx_slice(c):
        return idx_all.at[pl.ds(c * CHUNK, CHUNK)]

    def out_slice(c):
        return out_hbm.at[pl.ds(base + c * CHUNK, CHUNK)]

    pltpu.sync_copy(idx_hbm.at[pl.ds(base, EDGES_PER_W)], idx_all)

    def g_start(c, b):
        pltpu.async_copy(table_hbm.at[idx_slice(c)], rows[b], gsems[b])

    def g_wait(c, b):
        pltpu.make_async_copy(table_hbm.at[idx_slice(c)], rows[b], gsems[b]).wait()

    def s_start(c, b):
        pltpu.async_copy(rows[b], out_slice(c), osems[b])

    def s_wait(c, b):
        pltpu.make_async_copy(rows[b], out_slice(c), osems[b]).wait()

    g_start(0, 0)
    g_start(1, 1)
    g_wait(0, 0)
    s_start(0, 0)
    g_start(2, 2)
    g_wait(1, 1)
    s_start(1, 1)
    s_wait(0, 0)
    g_start(3, 0)

    def triple(j, carry):
        c = 3 * j + 2
        g_wait(c, 2)
        s_start(c, 2)
        s_wait(c - 1, 1)
        g_start(c + 2, 1)
        g_wait(c + 1, 0)
        s_start(c + 1, 0)
        s_wait(c, 2)
        g_start(c + 3, 2)
        g_wait(c + 2, 1)
        s_start(c + 2, 1)
        s_wait(c + 1, 0)
        g_start(c + 4, 0)
        return carry

    lax.fori_loop(0, N_TRIPLES - 1, triple, 0)

    c = N_CHUNKS - 3
    g_wait(c, 2)
    s_start(c, 2)
    s_wait(c - 1, 1)
    g_start(c + 2, 1)
    g_wait(c + 1, 0)
    s_start(c + 1, 0)
    s_wait(c, 2)
    g_wait(c + 2, 1)
    s_start(c + 2, 1)
    s_wait(c + 1, 0)
    s_wait(c + 2, 1)


def _sc_gather(table, heads):
    mesh = plsc.VectorSubcoreMesh(core_axis_name="c", subcore_axis_name="s")
    k = functools.partial(
        pl.kernel,
        mesh=mesh,
        out_type=jax.ShapeDtypeStruct((N_EDGES, D), jnp.float32),
        scratch_types=[
            pltpu.VMEM((EDGES_PER_W,), jnp.int32),
            [
                pltpu.VMEM((CHUNK, D), jnp.float32),
                pltpu.VMEM((CHUNK, D), jnp.float32),
                pltpu.VMEM((CHUNK, D), jnp.float32),
            ],
            [
                pltpu.SemaphoreType.DMA,
                pltpu.SemaphoreType.DMA,
                pltpu.SemaphoreType.DMA,
            ],
            [
                pltpu.SemaphoreType.DMA,
                pltpu.SemaphoreType.DMA,
                pltpu.SemaphoreType.DMA,
            ],
        ],
    )(_sc_gather_body)
    return k(table, heads)


def _add_mm_kernel(g_ref, e_ref, w_ref, o_ref):
    w2 = w_ref[:, D:]
    o_ref[...] = g_ref[...] + lax.dot_general(
        e_ref[...], w2, (((1,), (1,)), ((), ())),
        preferred_element_type=jnp.float32,
    )


def _edge_transform(G, E, W, blk):
    n_blocks = N_EDGES // blk
    return pl.pallas_call(
        _add_mm_kernel,
        grid=(n_blocks,),
        in_specs=[
            pl.BlockSpec((blk, D), lambda i: (i, 0)),
            pl.BlockSpec((blk, D), lambda i: (i, 0)),
            pl.BlockSpec((D, 2 * D), lambda i: (0, 0)),
        ],
        out_specs=pl.BlockSpec((blk, D), lambda i: (i, 0)),
        out_shape=jax.ShapeDtypeStruct((N_EDGES, D), jnp.float32),
    )(G, E, W)


@jax.jit
def kernel(H, E, heads, queries, W, b):
    b2d = b.reshape(1, D)
    T = _node_transform(H, W, b2d)
    G = _sc_gather(T, heads.astype(jnp.int32))
    return _edge_transform(G, E, W, blk=8000)
